# Initial kernel scaffold; baseline (speedup 1.0000x reference)
import jax
import jax.numpy as jnp
from jax.experimental import pallas as pl


def kernel(x, edge_index, W0, b0, g0, be0, W1, b1, g1, be1, W2, b2, g2, be2):
    raise NotImplementedError("write your pallas kernel here")


# speedup vs baseline: 6.1567x; 6.1567x over previous
import functools

import jax
import jax.numpy as jnp
from jax import lax
from jax.experimental import pallas as pl
from jax.experimental.pallas import tpu as pltpu
from jax.experimental.pallas import tpu_sc as plsc

N = 10000
E = 320000
D = 128
ALPHA = 0.1
EPS = 1e-5

NW = 32
N_PAD = 10240
RT = N_PAD // 16
EB = 128
E_PAD = ((E + NW * EB - 1) // (NW * EB)) * (NW * EB)
EW = E_PAD // NW
NBATCH = EW // EB

BR = 512
NB = N_PAD // BR

_mesh = plsc.VectorSubcoreMesh(core_axis_name="c", subcore_axis_name="s")


@functools.partial(
    pl.kernel,
    out_type=jax.ShapeDtypeStruct((2, N_PAD, D), jnp.float32),
    mesh=_mesh,
    scratch_types=[
        pltpu.VMEM_SHARED((N_PAD, D), jnp.float32),
        pltpu.VMEM((EB,), jnp.int32),
        pltpu.VMEM((EB, D), jnp.float32),
    ],
)
def _sc_degree(dst_hbm, zeros_hbm, ones_hbm, out_hbm, acc, didx, ones_v):
    c = lax.axis_index("c")
    s = lax.axis_index("s")
    wid = s * 2 + c
    r0 = s * RT
    pltpu.sync_copy(zeros_hbm.at[pl.ds(r0, RT)], acc.at[pl.ds(r0, RT)])
    pltpu.sync_copy(ones_hbm, ones_v)
    plsc.subcore_barrier()
    base = wid * EW

    def body(b, carry):
        off = base + b * EB
        pltpu.sync_copy(dst_hbm.at[pl.ds(off, EB)], didx)
        pltpu.sync_copy(ones_v, acc.at[didx], add=True)
        return carry

    lax.fori_loop(0, NBATCH, body, 0)
    plsc.subcore_barrier()
    pltpu.sync_copy(acc.at[pl.ds(r0, RT)], out_hbm.at[c].at[pl.ds(r0, RT)])


@functools.partial(
    pl.kernel,
    out_type=jax.ShapeDtypeStruct((2, N_PAD, D), jnp.float32),
    mesh=_mesh,
    scratch_types=[
        pltpu.VMEM_SHARED((N_PAD, D), jnp.float32),
        pltpu.VMEM((EB,), jnp.int32),
        pltpu.VMEM((EB,), jnp.int32),
        pltpu.VMEM((EB, D), jnp.float32),
        pltpu.SemaphoreType.DMA,
    ],
)
def _sc_propagate(y_hbm, z_hbm, src_hbm, dst_hbm, out_hbm, acc, sidx, didx, rows, sem):
    c = lax.axis_index("c")
    s = lax.axis_index("s")
    wid = s * 2 + c
    r0 = s * RT

    @pl.when(c == 0)
    def _():
        pltpu.sync_copy(y_hbm.at[pl.ds(r0, RT)], acc.at[pl.ds(r0, RT)])

    @pl.when(c == 1)
    def _():
        pltpu.sync_copy(z_hbm.at[pl.ds(r0, RT)], acc.at[pl.ds(r0, RT)])

    plsc.subcore_barrier()
    base = wid * EW

    def body(b, carry):
        off = base + b * EB
        pltpu.sync_copy(src_hbm.at[pl.ds(off, EB)], sidx)
        pltpu.sync_copy(dst_hbm.at[pl.ds(off, EB)], didx)
        pltpu.async_copy(y_hbm.at[sidx], rows, sem).wait()
        pltpu.sync_copy(rows, acc.at[didx], add=True)
        return carry

    lax.fori_loop(0, NBATCH, body, 0)
    plsc.subcore_barrier()
    pltpu.sync_copy(acc.at[pl.ds(r0, RT)], out_hbm.at[c].at[pl.ds(r0, RT)])


def _erf(x):
    a1, a2, a3, a4, a5 = 0.254829592, -0.284496736, 1.421413741, -1.453152027, 1.061405429
    p = 0.3275911
    sgn = jnp.sign(x)
    ax = jnp.abs(x)
    t = 1.0 / (1.0 + p * ax)
    y = 1.0 - (((((a5 * t + a4) * t + a3) * t + a2) * t + a1) * t) * jnp.exp(-ax * ax)
    return sgn * y


def _prep_body(degp_ref, xp_ref, dinv_ref, y_ref):
    i = pl.program_id(0)
    deg = degp_ref[0, :, 0:1] + degp_ref[1, :, 0:1] + 1.0
    rows = i * BR + lax.broadcasted_iota(jnp.int32, (BR, 1), 0)
    dinv = jnp.where(rows < N, lax.rsqrt(jnp.maximum(deg, 1.0)), 0.0)
    dinv_ref[...] = dinv
    y_ref[...] = xp_ref[...] * dinv


def _combine_body(p_ref, dinv_ref, s_ref, y_ref, sout_ref):
    z = p_ref[0] + p_ref[1]
    dinv = dinv_ref[...]
    xk = z * dinv
    sout_ref[...] = s_ref[...] + xk
    y_ref[...] = xk * dinv


def _heads_body(xp_ref, s2_ref, s4_ref, s8_ref, w_ref, b_ref,
                lin_ref, sum_ref, sq_ref):
    i = pl.program_id(0)

    @pl.when(i == 0)
    def _():
        sum_ref[...] = jnp.zeros_like(sum_ref)
        sq_ref[...] = jnp.zeros_like(sq_ref)

    xb = xp_ref[...]
    rows = i * BR + lax.broadcasted_iota(jnp.int32, (BR, 1), 0)
    msk = (rows < N).astype(jnp.float32)
    coefs = [(s2_ref, (1.0 - ALPHA) / 2.0),
             (s4_ref, (1.0 - ALPHA) / 4.0),
             (s8_ref, (1.0 - ALPHA) / 8.0)]
    for h, (s_ref, ck) in enumerate(coefs):
        hb = ALPHA * xb + ck * s_ref[...]
        lin = lax.dot_general(hb, w_ref[h], (((1,), (1,)), ((), ())),
                              preferred_element_type=jnp.float32,
                              precision=lax.Precision.HIGHEST) + b_ref[h]
        lin_ref[h] = lin
        lm = lin * msk
        sum_ref[h] += jnp.sum(lm, axis=0, keepdims=True)
        sq_ref[h] += jnp.sum(lm * lm, axis=0, keepdims=True)


def _bngelu_body(lin_ref, sum_ref, sq_ref, g_ref, be_ref, out_ref):
    inv_sqrt2 = 0.7071067811865476
    for h in range(3):
        lin = lin_ref[h]
        mean = sum_ref[h] / N
        var = sq_ref[h] / N - mean * mean
        r = lax.rsqrt(var + EPS)
        o = (lin - mean) * r * g_ref[h] + be_ref[h]
        out_ref[h] = o * 0.5 * (1.0 + _erf(o * inv_sqrt2))


_prep = pl.pallas_call(
    _prep_body,
    grid=(NB,),
    in_specs=[
        pl.BlockSpec((2, BR, D), lambda i: (0, i, 0)),
        pl.BlockSpec((BR, D), lambda i: (i, 0)),
    ],
    out_specs=[
        pl.BlockSpec((BR, 1), lambda i: (i, 0)),
        pl.BlockSpec((BR, D), lambda i: (i, 0)),
    ],
    out_shape=[
        jax.ShapeDtypeStruct((N_PAD, 1), jnp.float32),
        jax.ShapeDtypeStruct((N_PAD, D), jnp.float32),
    ],
)

_combine = pl.pallas_call(
    _combine_body,
    grid=(NB,),
    in_specs=[
        pl.BlockSpec((2, BR, D), lambda i: (0, i, 0)),
        pl.BlockSpec((BR, 1), lambda i: (i, 0)),
        pl.BlockSpec((BR, D), lambda i: (i, 0)),
    ],
    out_specs=[
        pl.BlockSpec((BR, D), lambda i: (i, 0)),
        pl.BlockSpec((BR, D), lambda i: (i, 0)),
    ],
    out_shape=[
        jax.ShapeDtypeStruct((N_PAD, D), jnp.float32),
        jax.ShapeDtypeStruct((N_PAD, D), jnp.float32),
    ],
)

_heads = pl.pallas_call(
    _heads_body,
    grid=(NB,),
    in_specs=[
        pl.BlockSpec((BR, D), lambda i: (i, 0)),
        pl.BlockSpec((BR, D), lambda i: (i, 0)),
        pl.BlockSpec((BR, D), lambda i: (i, 0)),
        pl.BlockSpec((BR, D), lambda i: (i, 0)),
        pl.BlockSpec((3, D, D), lambda i: (0, 0, 0)),
        pl.BlockSpec((3, 1, D), lambda i: (0, 0, 0)),
    ],
    out_specs=[
        pl.BlockSpec((3, BR, D), lambda i: (0, i, 0)),
        pl.BlockSpec((3, 1, D), lambda i: (0, 0, 0)),
        pl.BlockSpec((3, 1, D), lambda i: (0, 0, 0)),
    ],
    out_shape=[
        jax.ShapeDtypeStruct((3, N_PAD, D), jnp.float32),
        jax.ShapeDtypeStruct((3, 1, D), jnp.float32),
        jax.ShapeDtypeStruct((3, 1, D), jnp.float32),
    ],
)

_bngelu = pl.pallas_call(
    _bngelu_body,
    grid=(NB,),
    in_specs=[
        pl.BlockSpec((3, BR, D), lambda i: (0, i, 0)),
        pl.BlockSpec((3, 1, D), lambda i: (0, 0, 0)),
        pl.BlockSpec((3, 1, D), lambda i: (0, 0, 0)),
        pl.BlockSpec((3, 1, D), lambda i: (0, 0, 0)),
        pl.BlockSpec((3, 1, D), lambda i: (0, 0, 0)),
    ],
    out_specs=pl.BlockSpec((3, BR, D), lambda i: (0, i, 0)),
    out_shape=jax.ShapeDtypeStruct((3, N_PAD, D), jnp.float32),
)


def kernel(x, edge_index, W0, b0, g0, be0, W1, b1, g1, be1, W2, b2, g2, be2):
    fill = jnp.full((E_PAD - E,), N, jnp.int32)
    srcp = jnp.concatenate([edge_index[0], fill])
    dstp = jnp.concatenate([edge_index[1], fill])
    xp = jnp.pad(x, ((0, N_PAD - N), (0, 0)))

    onesD = jnp.ones((EB, D), jnp.float32)
    zerosD = jnp.zeros((N_PAD, D), jnp.float32)

    degp = _sc_degree(dstp, zerosD, onesD)
    dinv, y = _prep(degp, xp)

    S = jnp.zeros((N_PAD, D), jnp.float32)
    snaps = {}
    for k in range(1, 9):
        P = _sc_propagate(y, zerosD, srcp, dstp)
        y, S = _combine(P, dinv, S)
        if k in (2, 4, 8):
            snaps[k] = S

    Wstack = jnp.stack([W0, W1, W2])
    bstack = jnp.stack([b0, b1, b2])[:, None, :]
    gstack = jnp.stack([g0, g1, g2])[:, None, :]
    bestack = jnp.stack([be0, be1, be2])[:, None, :]

    lin, sums, sqs = _heads(xp, snaps[2], snaps[4], snaps[8], Wstack, bstack)
    outs = _bngelu(lin, sums, sqs, gstack, bestack)
    return (outs[0, :N], outs[1, :N], outs[2, :N])
